# Initial kernel scaffold; baseline (speedup 1.0000x reference)
import jax
import jax.numpy as jnp
from jax.experimental import pallas as pl


def kernel(fmap1, fmap2, fmap3, depths):
    raise NotImplementedError("write your pallas kernel here")


# speedup vs baseline: 28.3274x; 28.3274x over previous
import jax
import jax.numpy as jnp
from jax.experimental import pallas as pl

_D = 10
_OUT = 224
_B = 2
_KMAX = 384


def _resize_matrix(n_in: int) -> jnp.ndarray:
    eye = jnp.eye(n_in, dtype=jnp.float32)
    return jax.image.resize(eye, (_OUT, n_in), method="bilinear")


def _rc_kernel(d_ref, f1_ref, f2_ref, f3_ref,
               a1_ref, a1t_ref, a2_ref, a2t_ref, a3_ref, a3t_ref,
               out_ref):
    depth = d_ref[...]
    d_min = jnp.min(depth)
    d_max = jnp.max(depth)
    step = (d_max - d_min) / _D

    masks = []
    counts = []
    for dd in range(_D):
        lo = d_min + dd * step
        if dd < _D - 1:
            hi = d_min + (dd + 1) * step
            m = jnp.logical_and(depth >= lo, depth < hi)
        else:
            m = depth >= lo
        mf = m.astype(jnp.float32)
        masks.append(mf)
        counts.append(jnp.maximum(jnp.sum(mf), 1e-6))
    mfull = jnp.stack(masks, axis=0)
    mflat = mfull.reshape(_D * _B * _OUT, _OUT)

    out_ref[...] = jnp.zeros(out_ref.shape, dtype=jnp.float32)

    for l, (f_ref, a_ref, at_ref) in enumerate(
            ((f1_ref, a1_ref, a1t_ref),
             (f2_ref, a2_ref, a2t_ref),
             (f3_ref, a3_ref, a3t_ref))):
        f = f_ref[...]
        a = a_ref[...]
        at = at_ref[...]
        c_dim, h = f.shape[1], f.shape[2]
        t = jnp.dot(mflat, a, preferred_element_type=jnp.float32)
        t = t.reshape(_D, _B, _OUT, h)
        cols = []
        for dd in range(_D):
            acc = None
            for b in range(_B):
                w_db = jnp.dot(at, t[dd, b],
                               preferred_element_type=jnp.float32)
                s = jnp.sum(f[b] * w_db[None, :, :], axis=(1, 2))
                acc = s if acc is None else acc + s
            cols.append((acc / counts[dd])[:, None])
        out_ref[l, :c_dim, :] = jnp.concatenate(cols, axis=1)


def kernel(fmap1, fmap2, fmap3, depths):
    d = depths[:, 0]
    mats = []
    for f in (fmap1, fmap2, fmap3):
        a = _resize_matrix(f.shape[2])
        mats.extend([a, a.T])
    out = pl.pallas_call(
        _rc_kernel,
        out_shape=jax.ShapeDtypeStruct((3, _KMAX, _D), jnp.float32),
    )(d, fmap1, fmap2, fmap3, *mats)
    return out
